# Initial kernel scaffold; baseline (speedup 1.0000x reference)
import jax
import jax.numpy as jnp
from jax.experimental import pallas as pl


def kernel(x, y):
    raise NotImplementedError("write your pallas kernel here")


# speedup vs baseline: 24.6048x; 24.6048x over previous
import functools

import jax
import jax.numpy as jnp
from jax.experimental import pallas as pl
from jax.experimental.pallas import tpu as pltpu

_K = 16
_N = 4096
_R = 128
_NBLK = _N // _R


def _topk_mask(D, iota_i32):
    M = jnp.zeros(D.shape, jnp.float32)
    big_i = jnp.int32(2 * _N)
    for _ in range(_K):
        m = jnp.min(D, axis=1, keepdims=True)
        idxv = jnp.where(D == m, iota_i32, big_i)
        amin = jnp.min(idxv, axis=1, keepdims=True)
        h = (iota_i32 == amin).astype(jnp.float32)
        M = M + h
        D = D + h * jnp.float32(1e30)
    return M


def _acos(x):
    ax = jnp.abs(x)
    p = jnp.float32(-0.0012624911)
    p = p * ax + jnp.float32(0.0066700901)
    p = p * ax + jnp.float32(-0.0170881256)
    p = p * ax + jnp.float32(0.0308918810)
    p = p * ax + jnp.float32(-0.0501743046)
    p = p * ax + jnp.float32(0.0889789874)
    p = p * ax + jnp.float32(-0.2145988016)
    p = p * ax + jnp.float32(1.5707963050)
    a_pos = jnp.sqrt(jnp.maximum(1.0 - ax, 0.0)) * p
    return jnp.where(x >= 0, a_pos, jnp.float32(3.14159265358979) - a_pos)


def _eigen_ratio_block(S):
    k = jnp.float32(_K)
    mx = S[:, 0:1] / k
    my = S[:, 1:2] / k
    mz = S[:, 2:3] / k
    cxx = S[:, 3:4] / k - mx * mx
    cyy = S[:, 4:5] / k - my * my
    czz = S[:, 5:6] / k - mz * mz
    cxy = S[:, 6:7] / k - mx * my
    cxz = S[:, 7:8] / k - mx * mz
    cyz = S[:, 8:9] / k - my * mz

    q = (cxx + cyy + czz) * jnp.float32(1.0 / 3.0)
    p1 = cxy * cxy + cxz * cxz + cyz * cyz
    dxx = cxx - q
    dyy = cyy - q
    dzz = czz - q
    p2 = dxx * dxx + dyy * dyy + dzz * dzz + 2.0 * p1
    eps = jnp.float32(1e-30)
    safe = p2 > eps
    p = jnp.sqrt(jnp.maximum(p2, eps) * jnp.float32(1.0 / 6.0))
    inv_p = 1.0 / p
    b00 = dxx * inv_p
    b11 = dyy * inv_p
    b22 = dzz * inv_p
    b01 = cxy * inv_p
    b02 = cxz * inv_p
    b12 = cyz * inv_p
    detb = (b00 * (b11 * b22 - b12 * b12)
            - b01 * (b01 * b22 - b12 * b02)
            + b02 * (b01 * b12 - b11 * b02))
    r = jnp.clip(detb * 0.5, -1.0, 1.0)
    phi = _acos(r) * jnp.float32(1.0 / 3.0)
    e0 = q + 2.0 * p * jnp.cos(phi)
    e2 = q + 2.0 * p * jnp.cos(phi + jnp.float32(2.0943951023931953))
    e1 = 3.0 * q - e0 - e2
    return jnp.where(safe, e0 / e1, jnp.float32(1.0))


def _body(px_ref, pxT_ref, py_ref, pyT_ref, out_ref, er1_ref, er2_ref):
    b = pl.program_id(0)

    @pl.when(b == 0)
    def _():
        out_ref[:, :] = jnp.zeros((1, 1), jnp.float32)

    iota_i32 = jax.lax.broadcasted_iota(jnp.int32, (_R, _N), 1)

    def er_phase(p_ref, pT_ref, er_ref):
        pT = pT_ref[0]
        pts2 = jnp.sum(pT * pT, axis=0, keepdims=True)
        pT16 = pT.astype(jnp.bfloat16)

        def blk(i, _):
            r0 = i * _R
            a = p_ref[0, pl.ds(r0, _R), 0:3].astype(jnp.bfloat16)
            D = pts2 - 2.0 * jnp.dot(a, pT16, preferred_element_type=jnp.float32)
            M = _topk_mask(D, iota_i32)
            S = jnp.dot(M, p_ref[0], preferred_element_type=jnp.float32, precision=jax.lax.Precision.HIGHEST)
            er_ref[pl.ds(r0, _R), :] = _eigen_ratio_block(S)
            return 0

        jax.lax.fori_loop(0, _NBLK, blk, 0)

    er_phase(px_ref, pxT_ref, er1_ref)
    er_phase(py_ref, pyT_ref, er2_ref)

    def cross_phase(pa_ref, pbT_ref, era_ref, erb_ref):
        pbT = pbT_ref[0]
        pts2 = jnp.sum(pbT * pbT, axis=0, keepdims=True)
        pbT16 = pbT.astype(jnp.bfloat16)
        erb = erb_ref[:, :]

        def blk(i, sse):
            r0 = i * _R
            a = pa_ref[0, pl.ds(r0, _R), 0:3].astype(jnp.bfloat16)
            D = pts2 - 2.0 * jnp.dot(a, pbT16, preferred_element_type=jnp.float32)
            m = jnp.min(D, axis=1, keepdims=True)
            idxv = jnp.where(D == m, iota_i32, jnp.int32(2 * _N))
            amin = jnp.min(idxv, axis=1, keepdims=True)
            h = (iota_i32 == amin).astype(jnp.float32)
            corr = jnp.dot(h, erb, preferred_element_type=jnp.float32, precision=jax.lax.Precision.HIGHEST)
            d = era_ref[pl.ds(r0, _R), :] - corr
            return sse + jnp.sum(d * d, axis=(0, 1), keepdims=True)

        return jax.lax.fori_loop(0, _NBLK, blk, jnp.zeros((1, 1), jnp.float32))

    sse_x = cross_phase(px_ref, pyT_ref, er1_ref, er2_ref)
    sse_y = cross_phase(py_ref, pxT_ref, er2_ref, er1_ref)
    out_ref[:, :] += sse_x + sse_y


def _moments(pts):
    x = pts[..., 0:1]
    y = pts[..., 1:2]
    z = pts[..., 2:3]
    zeros = jnp.zeros(pts.shape[:-1] + (7,), pts.dtype)
    return jnp.concatenate(
        [x, y, z, x * x, y * y, z * z, x * y, x * z, y * z, zeros], axis=-1)


@jax.jit
def kernel(x, y):
    x3 = x[..., :3].astype(jnp.float32)
    y3 = y[..., :3].astype(jnp.float32)
    px = _moments(x3)
    py = _moments(y3)
    pxT = jnp.swapaxes(x3, 1, 2)
    pyT = jnp.swapaxes(y3, 1, 2)

    bspec_p = pl.BlockSpec((1, _N, 16), lambda b: (b, 0, 0))
    bspec_t = pl.BlockSpec((1, 3, _N), lambda b: (b, 0, 0))
    acc = pl.pallas_call(
        _body,
        grid=(x.shape[0],),
        in_specs=[bspec_p, bspec_t, bspec_p, bspec_t],
        out_specs=pl.BlockSpec((1, 1), lambda b: (0, 0)),
        out_shape=jax.ShapeDtypeStruct((1, 1), jnp.float32),
        scratch_shapes=[pltpu.VMEM((_N, 1), jnp.float32),
                        pltpu.VMEM((_N, 1), jnp.float32)],
    )(px, pxT, py, pyT)
    return acc[0, 0] * jnp.float32(0.5 / (_N * x.shape[0]))


# speedup vs baseline: 37.7965x; 1.5361x over previous
import functools

import jax
import jax.numpy as jnp
from jax.experimental import pallas as pl
from jax.experimental.pallas import tpu as pltpu

_K = 16
_N = 4096
_R = 128
_NBLK = _N // _R


def _topk_mask(D):
    M = jnp.zeros(D.shape, jnp.float32)
    big = jnp.float32(1e30)
    for t in range(_K):
        m = jnp.min(D, axis=1, keepdims=True)
        hb = D == m
        M = M + hb.astype(jnp.float32)
        if t + 1 < _K:
            D = jnp.where(hb, big, D)
    return M


def _acos(x):
    ax = jnp.abs(x)
    p = jnp.float32(-0.0012624911)
    p = p * ax + jnp.float32(0.0066700901)
    p = p * ax + jnp.float32(-0.0170881256)
    p = p * ax + jnp.float32(0.0308918810)
    p = p * ax + jnp.float32(-0.0501743046)
    p = p * ax + jnp.float32(0.0889789874)
    p = p * ax + jnp.float32(-0.2145988016)
    p = p * ax + jnp.float32(1.5707963050)
    a_pos = jnp.sqrt(jnp.maximum(1.0 - ax, 0.0)) * p
    return jnp.where(x >= 0, a_pos, jnp.float32(3.14159265358979) - a_pos)


def _eigen_ratio_block(S):
    k = jnp.float32(_K)
    mx = S[:, 0:1] / k
    my = S[:, 1:2] / k
    mz = S[:, 2:3] / k
    cxx = S[:, 3:4] / k - mx * mx
    cyy = S[:, 4:5] / k - my * my
    czz = S[:, 5:6] / k - mz * mz
    cxy = S[:, 6:7] / k - mx * my
    cxz = S[:, 7:8] / k - mx * mz
    cyz = S[:, 8:9] / k - my * mz

    q = (cxx + cyy + czz) * jnp.float32(1.0 / 3.0)
    p1 = cxy * cxy + cxz * cxz + cyz * cyz
    dxx = cxx - q
    dyy = cyy - q
    dzz = czz - q
    p2 = dxx * dxx + dyy * dyy + dzz * dzz + 2.0 * p1
    eps = jnp.float32(1e-30)
    safe = p2 > eps
    p = jnp.sqrt(jnp.maximum(p2, eps) * jnp.float32(1.0 / 6.0))
    inv_p = 1.0 / p
    b00 = dxx * inv_p
    b11 = dyy * inv_p
    b22 = dzz * inv_p
    b01 = cxy * inv_p
    b02 = cxz * inv_p
    b12 = cyz * inv_p
    detb = (b00 * (b11 * b22 - b12 * b12)
            - b01 * (b01 * b22 - b12 * b02)
            + b02 * (b01 * b12 - b11 * b02))
    r = jnp.clip(detb * 0.5, -1.0, 1.0)
    phi = _acos(r) * jnp.float32(1.0 / 3.0)
    e0 = q + 2.0 * p * jnp.cos(phi)
    e2 = q + 2.0 * p * jnp.cos(phi + jnp.float32(2.0943951023931953))
    e1 = 3.0 * q - e0 - e2
    return jnp.where(safe, e0 / e1, jnp.float32(1.0))


def _body(px_ref, pxT_ref, py_ref, pyT_ref, out_ref, er1_ref, er2_ref):
    b = pl.program_id(0)

    @pl.when(b == 0)
    def _():
        out_ref[:, :] = jnp.zeros((1, 1), jnp.float32)

    iota_i32 = jax.lax.broadcasted_iota(jnp.int32, (_R, _N), 1)

    def er_phase(p_ref, pT_ref, er_ref):
        pT = pT_ref[0]
        pts2 = jnp.sum(pT * pT, axis=0, keepdims=True)
        pT16 = pT.astype(jnp.bfloat16)

        def blk(i, _):
            r0 = i * _R
            a = p_ref[0, pl.ds(r0, _R), 0:3].astype(jnp.bfloat16)
            D = pts2 - 2.0 * jnp.dot(a, pT16, preferred_element_type=jnp.float32)
            M = _topk_mask(D)
            S = jnp.dot(M, p_ref[0], preferred_element_type=jnp.float32, precision=jax.lax.Precision.HIGHEST)
            er_ref[pl.ds(r0, _R), :] = _eigen_ratio_block(S)
            return 0

        jax.lax.fori_loop(0, _NBLK, blk, 0)

    er_phase(px_ref, pxT_ref, er1_ref)
    er_phase(py_ref, pyT_ref, er2_ref)

    def cross_phase(pa_ref, pbT_ref, era_ref, erb_ref):
        pbT = pbT_ref[0]
        pts2 = jnp.sum(pbT * pbT, axis=0, keepdims=True)
        pbT16 = pbT.astype(jnp.bfloat16)
        erb = erb_ref[:, :]

        def blk(i, sse):
            r0 = i * _R
            a = pa_ref[0, pl.ds(r0, _R), 0:3].astype(jnp.bfloat16)
            D = pts2 - 2.0 * jnp.dot(a, pbT16, preferred_element_type=jnp.float32)
            m = jnp.min(D, axis=1, keepdims=True)
            idxv = jnp.where(D == m, iota_i32, jnp.int32(2 * _N))
            amin = jnp.min(idxv, axis=1, keepdims=True)
            h = (iota_i32 == amin).astype(jnp.float32)
            corr = jnp.dot(h, erb, preferred_element_type=jnp.float32, precision=jax.lax.Precision.HIGHEST)
            d = era_ref[pl.ds(r0, _R), :] - corr
            return sse + jnp.sum(d * d, axis=(0, 1), keepdims=True)

        return jax.lax.fori_loop(0, _NBLK, blk, jnp.zeros((1, 1), jnp.float32))

    sse_x = cross_phase(px_ref, pyT_ref, er1_ref, er2_ref)
    sse_y = cross_phase(py_ref, pxT_ref, er2_ref, er1_ref)
    out_ref[:, :] += sse_x + sse_y


def _moments(pts):
    x = pts[..., 0:1]
    y = pts[..., 1:2]
    z = pts[..., 2:3]
    zeros = jnp.zeros(pts.shape[:-1] + (7,), pts.dtype)
    return jnp.concatenate(
        [x, y, z, x * x, y * y, z * z, x * y, x * z, y * z, zeros], axis=-1)


@jax.jit
def kernel(x, y):
    x3 = x[..., :3].astype(jnp.float32)
    y3 = y[..., :3].astype(jnp.float32)
    px = _moments(x3)
    py = _moments(y3)
    pxT = jnp.swapaxes(x3, 1, 2)
    pyT = jnp.swapaxes(y3, 1, 2)

    bspec_p = pl.BlockSpec((1, _N, 16), lambda b: (b, 0, 0))
    bspec_t = pl.BlockSpec((1, 3, _N), lambda b: (b, 0, 0))
    acc = pl.pallas_call(
        _body,
        grid=(x.shape[0],),
        in_specs=[bspec_p, bspec_t, bspec_p, bspec_t],
        out_specs=pl.BlockSpec((1, 1), lambda b: (0, 0)),
        out_shape=jax.ShapeDtypeStruct((1, 1), jnp.float32),
        scratch_shapes=[pltpu.VMEM((_N, 1), jnp.float32),
                        pltpu.VMEM((_N, 1), jnp.float32)],
    )(px, pxT, py, pyT)
    return acc[0, 0] * jnp.float32(0.5 / (_N * x.shape[0]))
